# baseline (speedup 1.0000x reference)
import jax
import jax.numpy as jnp
from jax.experimental import pallas as pl
from jax.experimental.pallas import tpu as pltpu

B, L, D = 4, 512, 140
U, T, NB = 4, 10, 2
DP = 144
NP = 2 * U + 2
IN_PIECES = (0, 1, 2, 3, 8)
OUT_PIECES = (4, 5, 6, 7, 9)
KF = 2 * (U + 1) * L


def _gcnn_kernel(nodes_ref, adj_ref, wioT_ref, wgT_ref, out_ref, kflat_ref):
    bf = jnp.bfloat16
    adjb = adj_ref[0]

    rare = ((adjb[U] + adjb[U + 1]) + (adjb[U + 2] + adjb[U + 3])) \
        + (adjb[U + 4] + adjb[U + 5])
    half = (U + 1) * L
    for t in range(U):
        a_bf = adjb[t].astype(bf)
        kflat_ref[t * L:(t + 1) * L, :] = a_bf.T
        kflat_ref[half + t * L:half + (t + 1) * L, :] = a_bf
    rare_bf = rare.astype(bf)
    kflat_ref[U * L:(U + 1) * L, :] = rare_bf.T
    kflat_ref[half + U * L:half + (U + 1) * L, :] = rare_bf

    hT = nodes_ref[0].T
    ones_row = jnp.ones((1, L), dtype=bf)

    for l in range(NB):
        aug = jnp.concatenate([hT.astype(bf), ones_row], axis=0)
        pT = jnp.dot(wioT_ref[l], aug,
                     preferred_element_type=jnp.float32).astype(bf)
        gT = jax.nn.sigmoid(
            jnp.dot(wgT_ref[l], aug, preferred_element_type=jnp.float32)
        ).astype(bf)

        gated = jnp.concatenate(
            [pT[DP * p:DP * p + D, :] * gT[p:p + 1, :]
             for p in IN_PIECES + OUT_PIECES], axis=1)
        msgT = jnp.dot(gated, kflat_ref[...],
                       preferred_element_type=jnp.float32)
        hT = jnp.maximum(msgT, 0.0) + hT

    out_ref[0] = hT.T


def kernel(nodes_embed, adj, Win_w, Win_b, Wout_w, Wout_b, Wing_w, Wing_b,
           Woutg_w, Woutg_b, Rin_w, Rin_b, Rout_w, Rout_b, Ring_w, Ring_b,
           Routg_w, Routg_b):
    w_pieces = [Win_w[:, :, t * D:(t + 1) * D] for t in range(U)] \
        + [Wout_w[:, :, t * D:(t + 1) * D] for t in range(U)] \
        + [Rin_w, Rout_w]
    b_pieces = [Win_b[:, t * D:(t + 1) * D] for t in range(U)] \
        + [Wout_b[:, t * D:(t + 1) * D] for t in range(U)] \
        + [Rin_b, Rout_b]
    blocks = []
    for wp, bp in zip(w_pieces, b_pieces):
        blk = jnp.concatenate([wp.transpose(0, 2, 1), bp[:, :, None]], axis=2)
        blocks.append(jnp.pad(blk, ((0, 0), (0, DP - D), (0, 0))))
    wioT = jnp.concatenate(blocks, axis=1).astype(jnp.bfloat16)

    wg = jnp.concatenate([Wing_w, Woutg_w, Ring_w, Routg_w], axis=2)
    bg = jnp.concatenate([Wing_b, Woutg_b, Ring_b, Routg_b], axis=1)
    wgT = jnp.concatenate([wg.transpose(0, 2, 1), bg[:, :, None]],
                          axis=2).astype(jnp.bfloat16)

    return pl.pallas_call(
        _gcnn_kernel,
        grid=(B,),
        in_specs=[
            pl.BlockSpec((1, L, D), lambda b: (b, 0, 0)),
            pl.BlockSpec((1, T, L, L), lambda b: (b, 0, 0, 0)),
            pl.BlockSpec((NB, NP * DP, D + 1), lambda b: (0, 0, 0)),
            pl.BlockSpec((NB, NP, D + 1), lambda b: (0, 0, 0)),
        ],
        out_specs=pl.BlockSpec((1, L, D), lambda b: (b, 0, 0)),
        out_shape=jax.ShapeDtypeStruct((B, L, D), jnp.float32),
        scratch_shapes=[
            pltpu.VMEM((KF, L), jnp.bfloat16),
        ],
        compiler_params=pltpu.CompilerParams(
            dimension_semantics=("arbitrary",),
            vmem_limit_bytes=100 * 1024 * 1024,
        ),
    )(nodes_embed, adj, wioT, wgT)
